# Initial kernel scaffold; baseline (speedup 1.0000x reference)
import jax
import jax.numpy as jnp
from jax.experimental import pallas as pl


def kernel(h, p, e, senders, receivers, snorm_n, WA, bA, WB, bB, WC, bC, WU, bU, WV, bV, WX, bX, WY, bY, gamma_e, beta_e, gamma_n, beta_n):
    raise NotImplementedError("write your pallas kernel here")


# speedup vs baseline: 1.8230x; 1.8230x over previous
import functools

import jax
import jax.numpy as jnp
from jax import lax
from jax.experimental import pallas as pl
from jax.experimental.pallas import tpu as pltpu
from jax.experimental.pallas import tpu_sc as plsc

N = 10000
E = 160000
D = 256
LANES = 16
NC = 2
NS = 16
NW = NC * NS
BAND = D // NC
GROUP = 128
NGROUPS = E // GROUP
RPT = 632
RPT_LAST = N - 15 * RPT
NBLK = 1000
EBLK = 2000


def _t1a_body(h_ref, p_ref, wa, ba, wb, bb, wu1, wu2, bu, wv1, wv2, bv,
              wx, bx, wy, by,
              a_out, b_out, u_out, x_out, v2_out, y2_out):
    h = h_ref[...]
    p = p_ref[...]
    f32 = jnp.float32
    a_out[...] = jnp.dot(h, wa[...], preferred_element_type=f32) + ba[...]
    b_out[...] = jnp.dot(h, wb[...], preferred_element_type=f32) + bb[...]
    u_out[...] = (jnp.dot(h, wu1[...], preferred_element_type=f32)
                  + jnp.dot(p, wu2[...], preferred_element_type=f32) + bu[...])
    x_out[...] = jnp.dot(p, wx[...], preferred_element_type=f32) + bx[...]
    v = (jnp.dot(h, wv1[...], preferred_element_type=f32)
         + jnp.dot(p, wv2[...], preferred_element_type=f32) + bv[...])
    y = jnp.dot(p, wy[...], preferred_element_type=f32) + by[...]
    for b in range(NC):
        v2_out[b] = v[:, b * BAND:(b + 1) * BAND]
        y2_out[b] = y[:, b * BAND:(b + 1) * BAND]


def _t1a(h, p, WA, bA, WB, bB, WU1, WU2, bU, WV1, WV2, bV, WX, bX, WY, bY):
    row = pl.BlockSpec((NBLK, D), lambda n: (n, 0))
    wspec = pl.BlockSpec((D, D), lambda n: (0, 0))
    bspec = pl.BlockSpec((1, D), lambda n: (0, 0))
    band = pl.BlockSpec((NC, NBLK, BAND), lambda n: (0, n, 0))
    f32 = jnp.float32
    return pl.pallas_call(
        _t1a_body,
        grid=(N // NBLK,),
        in_specs=[row, row, wspec, bspec, wspec, bspec, wspec, wspec, bspec,
                  wspec, wspec, bspec, wspec, bspec, wspec, bspec],
        out_specs=[row, row, row, row, band, band],
        out_shape=[jax.ShapeDtypeStruct((N, D), f32)] * 4
        + [jax.ShapeDtypeStruct((NC, N, BAND), f32)] * 2,
    )(h, p, WA, bA, WB, bB, WU1, WU2, bU, WV1, WV2, bV, WX, bX, WY, bY)


def _t1b_body(e_ref, wc, bc, c_out):
    c_out[...] = (jnp.dot(e_ref[...], wc[...], preferred_element_type=jnp.float32)
                  + bc[...])


def _t1b(e, WC, bC):
    return pl.pallas_call(
        _t1b_body,
        grid=(E // EBLK,),
        in_specs=[pl.BlockSpec((EBLK, D), lambda n: (n, 0)),
                  pl.BlockSpec((D, D), lambda n: (0, 0)),
                  pl.BlockSpec((1, D), lambda n: (0, 0))],
        out_specs=pl.BlockSpec((EBLK, D), lambda n: (n, 0)),
        out_shape=jax.ShapeDtypeStruct((E, D), jnp.float32),
    )(e, WC, bC)


def _t1c_body(eta_ref, s_out, q_out):
    x = eta_ref[...]
    ps = jnp.sum(x, axis=0, keepdims=True)
    pq = jnp.sum(x * x, axis=0, keepdims=True)

    @pl.when(pl.program_id(0) == 0)
    def _():
        s_out[...] = jnp.zeros_like(s_out)
        q_out[...] = jnp.zeros_like(q_out)

    s_out[...] += ps
    q_out[...] += pq


def _t1c(eta):
    f32 = jnp.float32
    acc = pl.BlockSpec((1, D), lambda n: (0, 0))
    return pl.pallas_call(
        _t1c_body,
        grid=(E // EBLK,),
        in_specs=[pl.BlockSpec((EBLK, D), lambda n: (n, 0))],
        out_specs=[acc, acc],
        out_shape=[jax.ShapeDtypeStruct((1, D), f32)] * 2,
    )(eta)


def _t1d_body(eta_ref, e_ref, sc_ref, sh_ref, eout_out, w_out):
    eo = e_ref[...] + jnp.maximum(eta_ref[...] * sc_ref[...] + sh_ref[...], 0.0)
    eout_out[...] = eo
    w_out[...] = 1.0 / (1.0 + jnp.exp(-eo))


def _t1d(eta, e, scale, shift):
    f32 = jnp.float32
    row = pl.BlockSpec((EBLK, D), lambda n: (n, 0))
    vec = pl.BlockSpec((1, D), lambda n: (0, 0))
    return pl.pallas_call(
        _t1d_body,
        grid=(E // EBLK,),
        in_specs=[row, row, vec, vec],
        out_specs=[row, row],
        out_shape=[jax.ShapeDtypeStruct((E, D), f32)] * 2,
    )(eta, e, scale, shift)


def _t2a_body(u_ref, agg_ref, wsum_ref, pagg_ref, x_ref, p_ref, sn_ref,
              npre_out, pout_out, s_out, q_out):
    agg = jnp.concatenate([agg_ref[b] for b in range(NC)], axis=1)
    wsum = jnp.concatenate([wsum_ref[b] for b in range(NC)], axis=1)
    pagg = jnp.concatenate([pagg_ref[b] for b in range(NC)], axis=1)
    inv = 1.0 / (wsum + 1e-6)
    nf = (u_ref[...] + agg * inv) * sn_ref[...]
    npre_out[...] = nf
    pout_out[...] = p_ref[...] + jnp.tanh(x_ref[...] + pagg * inv)

    @pl.when(pl.program_id(0) == 0)
    def _():
        s_out[...] = jnp.zeros_like(s_out)
        q_out[...] = jnp.zeros_like(q_out)

    s_out[...] += jnp.sum(nf, axis=0, keepdims=True)
    q_out[...] += jnp.sum(nf * nf, axis=0, keepdims=True)


def _t2a(U, aggB, wsumB, paggB, X, p, snorm2):
    f32 = jnp.float32
    row = pl.BlockSpec((NBLK, D), lambda n: (n, 0))
    band = pl.BlockSpec((NC, NBLK, BAND), lambda n: (0, n, 0))
    acc = pl.BlockSpec((1, D), lambda n: (0, 0))
    return pl.pallas_call(
        _t2a_body,
        grid=(N // NBLK,),
        in_specs=[row, band, band, band, row, row,
                  pl.BlockSpec((NBLK, 1), lambda n: (n, 0))],
        out_specs=[row, row, acc, acc],
        out_shape=[jax.ShapeDtypeStruct((N, D), f32)] * 2
        + [jax.ShapeDtypeStruct((1, D), f32)] * 2,
    )(U, aggB, wsumB, paggB, X, p, snorm2)


def _t2b_body(h_ref, npre_ref, sc_ref, sh_ref, hout_out):
    nf = jnp.maximum(npre_ref[...] * sc_ref[...] + sh_ref[...], 0.0)
    hout_out[...] = h_ref[...] + nf


def _t2b(h, npre, scale2, shift2):
    row = pl.BlockSpec((NBLK, D), lambda n: (n, 0))
    vec = pl.BlockSpec((1, D), lambda n: (0, 0))
    return pl.pallas_call(
        _t2b_body,
        grid=(N // NBLK,),
        in_specs=[row, row, vec, vec],
        out_specs=row,
        out_shape=jax.ShapeDtypeStruct((N, D), jnp.float32),
    )(h, npre, scale2, shift2)


@functools.cache
def _sc_mesh():
    return plsc.VectorSubcoreMesh(core_axis_name="c", subcore_axis_name="s")


def _s1_body(a_hbm, b_hbm, c_hbm, si_hbm, rj_hbm, eta_hbm,
             i_v, j_v, a_t, b_t, c_t, sem_a, sem_b):
    cid = lax.axis_index("c")
    sid = lax.axis_index("s")
    wid = sid * NC + cid

    def group_body(t, carry):
        g = t * NW + wid

        @pl.when(g < NGROUPS)
        def _():
            base = g * GROUP
            pltpu.sync_copy(si_hbm.at[pl.ds(base, GROUP)], i_v)
            pltpu.sync_copy(rj_hbm.at[pl.ds(base, GROUP)], j_v)
            cp_a = pltpu.async_copy(a_hbm.at[i_v], a_t, sem_a)
            cp_b = pltpu.async_copy(b_hbm.at[j_v], b_t, sem_b)
            pltpu.sync_copy(c_hbm.at[pl.ds(base, GROUP)], c_t)
            cp_a.wait()
            cp_b.wait()

            def row_body(r, c2):
                for u in range(D // LANES):
                    sl = pl.ds(u * LANES, LANES)
                    c_t[r, sl] = a_t[r, sl] + b_t[r, sl] + c_t[r, sl]
                return c2

            lax.fori_loop(0, GROUP, row_body, 0)
            pltpu.sync_copy(c_t, eta_hbm.at[pl.ds(base, GROUP)])

        return carry

    lax.fori_loop(0, (NGROUPS + NW - 1) // NW, group_body, 0)


@functools.cache
def _s1_kernel():
    return pl.kernel(
        _s1_body,
        mesh=_sc_mesh(),
        out_type=jax.ShapeDtypeStruct((E, D), jnp.float32),
        scratch_types=[
            pltpu.VMEM((GROUP,), jnp.int32),
            pltpu.VMEM((GROUP,), jnp.int32),
            pltpu.VMEM((GROUP, D), jnp.float32),
            pltpu.VMEM((GROUP, D), jnp.float32),
            pltpu.VMEM((GROUP, D), jnp.float32),
            pltpu.SemaphoreType.DMA,
            pltpu.SemaphoreType.DMA,
        ],
    )


def _s1(A, B, C, senders, receivers):
    return _s1_kernel()(A, B, C, senders, receivers)


def _s2_body(w_hbm, si_hbm, rj_hbm, v2_hbm, y2_hbm,
             wsum_hbm, agg_hbm, pagg_hbm,
             i_v, j_v, jo_v, w_t, g_t, z_t, acc, sem_g):
    cid = lax.axis_index("c")
    sid = lax.axis_index("s")
    col0 = cid * BAND
    zero16 = jnp.zeros((LANES,), jnp.float32)

    def zrow(r, carry):
        for u in range(BAND // LANES):
            z_t[r, pl.ds(u * LANES, LANES)] = zero16
        return carry

    lax.fori_loop(0, GROUP, zrow, 0)

    def zero_acc():
        @pl.when(sid < 15)
        def _():
            base = sid * RPT
            for k in range(4):
                pltpu.sync_copy(z_t, acc.at[pl.ds(base + k * GROUP, GROUP)])
            pltpu.sync_copy(z_t.at[pl.ds(0, RPT - 4 * GROUP)],
                            acc.at[pl.ds(base + 4 * GROUP, RPT - 4 * GROUP)])

        @pl.when(sid == 15)
        def _():
            base = 15 * RPT
            for k in range(4):
                pltpu.sync_copy(z_t, acc.at[pl.ds(base + k * GROUP, GROUP)])
            pltpu.sync_copy(
                z_t.at[pl.ds(0, RPT_LAST - 4 * GROUP)],
                acc.at[pl.ds(base + 4 * GROUP, RPT_LAST - 4 * GROUP)])

    def flush_acc(dst_hbm):
        @pl.when(sid < 15)
        def _():
            ro = sid * RPT
            pltpu.sync_copy(acc.at[pl.ds(ro, RPT)],
                            dst_hbm.at[pl.ds(cid * N + ro, RPT)])

        @pl.when(sid == 15)
        def _():
            ro = 15 * RPT
            pltpu.sync_copy(acc.at[pl.ds(ro, RPT_LAST)],
                            dst_hbm.at[pl.ds(cid * N + ro, RPT_LAST)])

    def sweep(gather_hbm):
        def group_body(t, carry):
            g = t * NS + sid

            @pl.when(g < NGROUPS)
            def _():
                base = g * GROUP
                pltpu.sync_copy(si_hbm.at[pl.ds(base, GROUP)], i_v)
                if gather_hbm is not None:
                    pltpu.sync_copy(rj_hbm.at[pl.ds(base, GROUP)], j_v)
                    for q in range(GROUP // LANES):
                        sl = pl.ds(q * LANES, LANES)
                        jo_v[sl] = j_v[sl] + cid * N
                    cp = pltpu.async_copy(gather_hbm.at[jo_v], g_t, sem_g)
                    pltpu.sync_copy(
                        w_hbm.at[pl.ds(base, GROUP), pl.ds(col0, BAND)], w_t)
                    cp.wait()

                    def row_body(r, c2):
                        for u in range(BAND // LANES):
                            sl = pl.ds(u * LANES, LANES)
                            g_t[r, sl] = g_t[r, sl] * w_t[r, sl]
                        return c2

                    lax.fori_loop(0, GROUP, row_body, 0)
                    pltpu.sync_copy(g_t, acc.at[i_v], add=True)
                else:
                    pltpu.sync_copy(
                        w_hbm.at[pl.ds(base, GROUP), pl.ds(col0, BAND)], w_t)
                    pltpu.sync_copy(w_t, acc.at[i_v], add=True)

            return carry

        lax.fori_loop(0, (NGROUPS + NS - 1) // NS, group_body, 0)

    for gather_hbm, dst_hbm in ((None, wsum_hbm),
                                (v2_hbm, agg_hbm),
                                (y2_hbm, pagg_hbm)):
        zero_acc()
        plsc.subcore_barrier()
        sweep(gather_hbm)
        plsc.subcore_barrier()
        flush_acc(dst_hbm)
        plsc.subcore_barrier()


@functools.cache
def _s2_kernel():
    return pl.kernel(
        _s2_body,
        mesh=_sc_mesh(),
        out_type=[jax.ShapeDtypeStruct((NC * N, BAND), jnp.float32),
                  jax.ShapeDtypeStruct((NC * N, BAND), jnp.float32),
                  jax.ShapeDtypeStruct((NC * N, BAND), jnp.float32)],
        scratch_types=[
            pltpu.VMEM((GROUP,), jnp.int32),
            pltpu.VMEM((GROUP,), jnp.int32),
            pltpu.VMEM((GROUP,), jnp.int32),
            pltpu.VMEM((GROUP, BAND), jnp.float32),
            pltpu.VMEM((GROUP, BAND), jnp.float32),
            pltpu.VMEM((GROUP, BAND), jnp.float32),
            pltpu.VMEM_SHARED((N, BAND), jnp.float32),
            pltpu.SemaphoreType.DMA,
        ],
    )


def _s2(w, senders, receivers, V2, Y2):
    return _s2_kernel()(w, senders, receivers, V2, Y2)


def kernel(h, p, e, senders, receivers, snorm_n, WA, bA, WB, bB, WC, bC,
           WU, bU, WV, bV, WX, bX, WY, bY, gamma_e, beta_e, gamma_n, beta_n):
    bA2 = bA.reshape(1, D)
    bB2 = bB.reshape(1, D)
    bC2 = bC.reshape(1, D)
    bU2 = bU.reshape(1, D)
    bV2 = bV.reshape(1, D)
    bX2 = bX.reshape(1, D)
    bY2 = bY.reshape(1, D)
    WU1, WU2 = WU[:D], WU[D:]
    WV1, WV2 = WV[:D], WV[D:]

    A, B, U, X, V2, Y2 = _t1a(h, p, WA, bA2, WB, bB2, WU1, WU2, bU2,
                              WV1, WV2, bV2, WX, bX2, WY, bY2)
    C = _t1b(e, WC, bC2)
    eta = _s1(A, B, C, senders, receivers)

    s, q = _t1c(eta)
    mean = s / E
    var = q / E - mean * mean
    rs = lax.rsqrt(var + 1e-5)
    scale = gamma_e.reshape(1, D) * rs
    shift = beta_e.reshape(1, D) - mean * scale

    e_out, w = _t1d(eta, e, scale, shift)

    wsumB, aggB, paggB = _s2(w, senders, receivers,
                             V2.reshape(NC * N, BAND),
                             Y2.reshape(NC * N, BAND))

    snorm2 = snorm_n.reshape(N, 1)
    npre, p_out, ns, nq = _t2a(U, aggB.reshape(NC, N, BAND),
                               wsumB.reshape(NC, N, BAND),
                               paggB.reshape(NC, N, BAND), X, p, snorm2)
    mean2 = ns / N
    var2 = nq / N - mean2 * mean2
    rs2 = lax.rsqrt(var2 + 1e-5)
    scale2 = gamma_n.reshape(1, D) * rs2
    shift2 = beta_n.reshape(1, D) - mean2 * scale2
    h_out = _t2b(h, npre, scale2, shift2)

    return (h_out, p_out, e_out)


# speedup vs baseline: 2.3936x; 1.3130x over previous
import functools

import jax
import jax.numpy as jnp
from jax import lax
from jax.experimental import pallas as pl
from jax.experimental.pallas import tpu as pltpu
from jax.experimental.pallas import tpu_sc as plsc

N = 10000
E = 160000
D = 256
LANES = 16
NC = 2
NS = 16
NW = NC * NS
BAND = D // NC
DC = 64
NCHUNK = D // DC
GROUP = 128
NGROUPS = E // GROUP
RPT = 632
RPT_LAST = N - 15 * RPT
NBLK = 1000
EBLK = 2000


def _t1a_body(h_ref, p_ref, wa, ba, wb, bb, wu1, wu2, bu, wv1, wv2, bv,
              wx, bx, wy, by,
              a_out, b_out, u_out, x_out, vy4_out):
    h = h_ref[...]
    p = p_ref[...]
    f32 = jnp.float32
    a_out[...] = jnp.dot(h, wa[...], preferred_element_type=f32) + ba[...]
    b_out[...] = jnp.dot(h, wb[...], preferred_element_type=f32) + bb[...]
    u_out[...] = (jnp.dot(h, wu1[...], preferred_element_type=f32)
                  + jnp.dot(p, wu2[...], preferred_element_type=f32) + bu[...])
    x_out[...] = jnp.dot(p, wx[...], preferred_element_type=f32) + bx[...]
    v = (jnp.dot(h, wv1[...], preferred_element_type=f32)
         + jnp.dot(p, wv2[...], preferred_element_type=f32) + bv[...])
    y = jnp.dot(p, wy[...], preferred_element_type=f32) + by[...]
    for c in range(NCHUNK):
        vy4_out[c] = jnp.concatenate(
            [v[:, c * DC:(c + 1) * DC], y[:, c * DC:(c + 1) * DC]], axis=1)


def _t1a(h, p, WA, bA, WB, bB, WU1, WU2, bU, WV1, WV2, bV, WX, bX, WY, bY):
    row = pl.BlockSpec((NBLK, D), lambda n: (n, 0))
    wspec = pl.BlockSpec((D, D), lambda n: (0, 0))
    bspec = pl.BlockSpec((1, D), lambda n: (0, 0))
    vyspec = pl.BlockSpec((NCHUNK, NBLK, 2 * DC), lambda n: (0, n, 0))
    f32 = jnp.float32
    return pl.pallas_call(
        _t1a_body,
        grid=(N // NBLK,),
        in_specs=[row, row, wspec, bspec, wspec, bspec, wspec, wspec, bspec,
                  wspec, wspec, bspec, wspec, bspec, wspec, bspec],
        out_specs=[row, row, row, row, vyspec],
        out_shape=[jax.ShapeDtypeStruct((N, D), f32)] * 4
        + [jax.ShapeDtypeStruct((NCHUNK, N, 2 * DC), f32)],
    )(h, p, WA, bA, WB, bB, WU1, WU2, bU, WV1, WV2, bV, WX, bX, WY, bY)


def _t1b_body(e_ref, wc, bc, c_out):
    c_out[...] = (jnp.dot(e_ref[...], wc[...], preferred_element_type=jnp.float32)
                  + bc[...])


def _t1b(e, WC, bC):
    return pl.pallas_call(
        _t1b_body,
        grid=(E // EBLK,),
        in_specs=[pl.BlockSpec((EBLK, D), lambda n: (n, 0)),
                  pl.BlockSpec((D, D), lambda n: (0, 0)),
                  pl.BlockSpec((1, D), lambda n: (0, 0))],
        out_specs=pl.BlockSpec((EBLK, D), lambda n: (n, 0)),
        out_shape=jax.ShapeDtypeStruct((E, D), jnp.float32),
    )(e, WC, bC)


def _t1c_body(eta_ref, s_out, q_out):
    x = eta_ref[...]
    ps = jnp.sum(x, axis=0, keepdims=True)
    pq = jnp.sum(x * x, axis=0, keepdims=True)

    @pl.when(pl.program_id(0) == 0)
    def _():
        s_out[...] = jnp.zeros_like(s_out)
        q_out[...] = jnp.zeros_like(q_out)

    s_out[...] += ps
    q_out[...] += pq


def _t1c(eta):
    f32 = jnp.float32
    acc = pl.BlockSpec((1, D), lambda n: (0, 0))
    return pl.pallas_call(
        _t1c_body,
        grid=(E // EBLK,),
        in_specs=[pl.BlockSpec((EBLK, D), lambda n: (n, 0))],
        out_specs=[acc, acc],
        out_shape=[jax.ShapeDtypeStruct((1, D), f32)] * 2,
    )(eta)


def _t1d_body(eta_ref, e_ref, sc_ref, sh_ref, eout_out, w_out):
    eo = e_ref[...] + jnp.maximum(eta_ref[...] * sc_ref[...] + sh_ref[...], 0.0)
    eout_out[...] = eo
    w_out[...] = 1.0 / (1.0 + jnp.exp(-eo))


def _t1d(eta, e, scale, shift):
    f32 = jnp.float32
    row = pl.BlockSpec((EBLK, D), lambda n: (n, 0))
    vec = pl.BlockSpec((1, D), lambda n: (0, 0))
    return pl.pallas_call(
        _t1d_body,
        grid=(E // EBLK,),
        in_specs=[row, row, vec, vec],
        out_specs=[row, row],
        out_shape=[jax.ShapeDtypeStruct((E, D), f32)] * 2,
    )(eta, e, scale, shift)


def _t2a_body(u_ref, mp_ref, wsum_ref, x_ref, p_ref, sn_ref,
              npre_out, pout_out, s_out, q_out):
    agg = jnp.concatenate([mp_ref[c][:, :DC] for c in range(NCHUNK)], axis=1)
    pagg = jnp.concatenate([mp_ref[c][:, DC:] for c in range(NCHUNK)], axis=1)
    wsum = jnp.concatenate([wsum_ref[b] for b in range(NC)], axis=1)
    inv = 1.0 / (wsum + 1e-6)
    nf = (u_ref[...] + agg * inv) * sn_ref[...]
    npre_out[...] = nf
    pout_out[...] = p_ref[...] + jnp.tanh(x_ref[...] + pagg * inv)

    @pl.when(pl.program_id(0) == 0)
    def _():
        s_out[...] = jnp.zeros_like(s_out)
        q_out[...] = jnp.zeros_like(q_out)

    s_out[...] += jnp.sum(nf, axis=0, keepdims=True)
    q_out[...] += jnp.sum(nf * nf, axis=0, keepdims=True)


def _t2a(U, mp4, wsumB, X, p, snorm2):
    f32 = jnp.float32
    row = pl.BlockSpec((NBLK, D), lambda n: (n, 0))
    mpspec = pl.BlockSpec((NCHUNK, NBLK, 2 * DC), lambda n: (0, n, 0))
    band = pl.BlockSpec((NC, NBLK, BAND), lambda n: (0, n, 0))
    acc = pl.BlockSpec((1, D), lambda n: (0, 0))
    return pl.pallas_call(
        _t2a_body,
        grid=(N // NBLK,),
        in_specs=[row, mpspec, band, row, row,
                  pl.BlockSpec((NBLK, 1), lambda n: (n, 0))],
        out_specs=[row, row, acc, acc],
        out_shape=[jax.ShapeDtypeStruct((N, D), f32)] * 2
        + [jax.ShapeDtypeStruct((1, D), f32)] * 2,
    )(U, mp4, wsumB, X, p, snorm2)


def _t2b_body(h_ref, npre_ref, sc_ref, sh_ref, hout_out):
    nf = jnp.maximum(npre_ref[...] * sc_ref[...] + sh_ref[...], 0.0)
    hout_out[...] = h_ref[...] + nf


def _t2b(h, npre, scale2, shift2):
    row = pl.BlockSpec((NBLK, D), lambda n: (n, 0))
    vec = pl.BlockSpec((1, D), lambda n: (0, 0))
    return pl.pallas_call(
        _t2b_body,
        grid=(N // NBLK,),
        in_specs=[row, row, vec, vec],
        out_specs=row,
        out_shape=jax.ShapeDtypeStruct((N, D), jnp.float32),
    )(h, npre, scale2, shift2)


@functools.cache
def _sc_mesh():
    return plsc.VectorSubcoreMesh(core_axis_name="c", subcore_axis_name="s")


def _s1_body(a_hbm, b_hbm, c_hbm, si_hbm, rj_hbm, eta_hbm,
             i_v, j_v, a_t, b_t, c_t, sem_a, sem_b):
    cid = lax.axis_index("c")
    sid = lax.axis_index("s")
    wid = sid * NC + cid

    def group_body(t, carry):
        g = t * NW + wid

        @pl.when(g < NGROUPS)
        def _():
            base = g * GROUP
            pltpu.sync_copy(si_hbm.at[pl.ds(base, GROUP)], i_v)
            pltpu.sync_copy(rj_hbm.at[pl.ds(base, GROUP)], j_v)
            cp_a = pltpu.async_copy(a_hbm.at[i_v], a_t, sem_a)
            cp_b = pltpu.async_copy(b_hbm.at[j_v], b_t, sem_b)
            pltpu.sync_copy(c_hbm.at[pl.ds(base, GROUP)], c_t)
            cp_a.wait()
            cp_b.wait()

            def row_body(r, c2):
                for u in range(D // LANES):
                    sl = pl.ds(u * LANES, LANES)
                    c_t[r, sl] = a_t[r, sl] + b_t[r, sl] + c_t[r, sl]
                return c2

            lax.fori_loop(0, GROUP, row_body, 0)
            pltpu.sync_copy(c_t, eta_hbm.at[pl.ds(base, GROUP)])

        return carry

    lax.fori_loop(0, (NGROUPS + NW - 1) // NW, group_body, 0)


@functools.cache
def _s1_kernel():
    return pl.kernel(
        _s1_body,
        mesh=_sc_mesh(),
        out_type=jax.ShapeDtypeStruct((E, D), jnp.float32),
        scratch_types=[
            pltpu.VMEM((GROUP,), jnp.int32),
            pltpu.VMEM((GROUP,), jnp.int32),
            pltpu.VMEM((GROUP, D), jnp.float32),
            pltpu.VMEM((GROUP, D), jnp.float32),
            pltpu.VMEM((GROUP, D), jnp.float32),
            pltpu.SemaphoreType.DMA,
            pltpu.SemaphoreType.DMA,
        ],
    )


def _s1(A, B, C, senders, receivers):
    return _s1_kernel()(A, B, C, senders, receivers)


GR2 = 64
NG2 = E // GR2
NT2 = 160
NTH = NT2 // 2
NGPAD = NT2 * NS


def _s2_body(w_hbm, si_hbm, rj_hbm, vy4_hbm,
             wsum_hbm, mp_hbm,
             i_all, j_all, io0, io1, jo0, jo1, wc0, wc1, g0, g1, acc,
             sem_g0, sem_g1, sem_w0, sem_w1):
    cid = lax.axis_index("c")
    sid = lax.axis_index("s")
    col0 = cid * BAND
    zero16 = jnp.zeros((LANES,), jnp.float32)

    def load_idx(lo_h):
        off = pl.multiple_of(lo_h * GR2, 8)
        pltpu.sync_copy(si_hbm.at[pl.ds(off, NTH * GR2)], i_all)
        pltpu.sync_copy(rj_hbm.at[pl.ds(off, NTH * GR2)], j_all)

    def stage_idx(t, io):
        for q in range(GR2 // LANES):
            sl = pl.ds(q * LANES, LANES)
            io[sl] = i_all[pl.ds(t * GR2 + q * LANES, LANES)]

    def zero_acc():
        def zrow(r, carry):
            for u in range(BAND // LANES):
                g0[r, pl.ds(u * LANES, LANES)] = zero16
            return carry

        lax.fori_loop(0, GR2, zrow, 0)

        @pl.when(sid < 15)
        def _():
            base = sid * RPT
            for k in range(9):
                pltpu.sync_copy(g0, acc.at[pl.ds(base + k * GR2, GR2)])
            pltpu.sync_copy(g0.at[pl.ds(0, RPT - 9 * GR2)],
                            acc.at[pl.ds(base + 9 * GR2, RPT - 9 * GR2)])

        @pl.when(sid == 15)
        def _():
            base = 15 * RPT
            for k in range(8):
                pltpu.sync_copy(g0, acc.at[pl.ds(base + k * GR2, GR2)])
            pltpu.sync_copy(
                g0.at[pl.ds(0, RPT_LAST - 8 * GR2)],
                acc.at[pl.ds(base + 8 * GR2, RPT_LAST - 8 * GR2)])

    def flush_acc(dst_hbm, dbase):
        @pl.when(sid < 15)
        def _():
            ro = sid * RPT
            pltpu.sync_copy(acc.at[pl.ds(ro, RPT)],
                            dst_hbm.at[pl.ds(dbase + ro, RPT)])

        @pl.when(sid == 15)
        def _():
            ro = 15 * RPT
            pltpu.sync_copy(acc.at[pl.ds(ro, RPT_LAST)],
                            dst_hbm.at[pl.ds(dbase + ro, RPT_LAST)])

    def _wband(lo_h, t):
        off = pl.multiple_of((lo_h + t) * GR2, 8)
        return w_hbm.at[pl.ds(off, GR2), pl.ds(col0, BAND)]

    def pipe(issue, proc):
        for hh in range(2):
            lo_h = sid * NT2 + hh * NTH
            cnt_h = jnp.minimum(NTH, NG2 - lo_h)
            load_idx(lo_h)

            @pl.when(0 < cnt_h)
            def _():
                issue(0, lo_h, 0)

            def pair(tp, carry):
                t0 = 2 * tp

                @pl.when(t0 + 1 < cnt_h)
                def _():
                    issue(t0 + 1, lo_h, 1)

                @pl.when(t0 < cnt_h)
                def _():
                    proc(t0, lo_h, 0)

                @pl.when(t0 + 2 < cnt_h)
                def _():
                    issue(t0 + 2, lo_h, 0)

                @pl.when(t0 + 1 < cnt_h)
                def _():
                    proc(t0 + 1, lo_h, 1)

                return carry

            lax.fori_loop(0, NTH // 2, pair, 0)

    gbuf = (g0, g1)
    wbuf = (wc0, wc1)
    iobuf = (io0, io1)
    jobuf = (jo0, jo1)
    sem_gs = (sem_g0, sem_g1)
    sem_ws = (sem_w0, sem_w1)

    def w_issue(t, lo_h, s):
        pltpu.async_copy(_wband(lo_h, t), gbuf[s], sem_ws[s])

    def w_proc(t, lo_h, s):
        pltpu.make_async_copy(_wband(lo_h, t), gbuf[s], sem_ws[s]).wait()
        stage_idx(t, iobuf[s])
        pltpu.sync_copy(gbuf[s], acc.at[iobuf[s]], add=True)

    zero_acc()
    plsc.subcore_barrier()
    pipe(w_issue, w_proc)
    plsc.subcore_barrier()
    flush_acc(wsum_hbm, cid * N)
    plsc.subcore_barrier()

    for ch in range(NCHUNK // NC):
        chunk = cid * (NCHUNK // NC) + ch

        def c_issue(t, lo_h, s, _ch=ch):
            jo = jobuf[s]
            for q in range(GR2 // LANES):
                sl = pl.ds(q * LANES, LANES)
                jo[sl] = (j_all[pl.ds(t * GR2 + q * LANES, LANES)]
                          + (cid * (NCHUNK // NC) + _ch) * N)
            pltpu.async_copy(vy4_hbm.at[jo], gbuf[s], sem_gs[s])
            pltpu.async_copy(_wband(lo_h, t), wbuf[s], sem_ws[s])

        def c_proc(t, lo_h, s, _ch=ch):
            g_t = gbuf[s]
            wc = wbuf[s]
            pltpu.make_async_copy(vy4_hbm.at[jobuf[s]], g_t,
                                  sem_gs[s]).wait()
            pltpu.make_async_copy(_wband(lo_h, t), wc, sem_ws[s]).wait()

            def row_body(r, c2):
                for u in range(DC // LANES):
                    sl = pl.ds(u * LANES, LANES)
                    sr = pl.ds(DC + u * LANES, LANES)
                    w = wc[r, pl.ds(_ch * DC + u * LANES, LANES)]
                    g_t[r, sl] = g_t[r, sl] * w
                    g_t[r, sr] = g_t[r, sr] * w
                return c2

            lax.fori_loop(0, GR2, row_body, 0)
            stage_idx(t, iobuf[s])
            pltpu.sync_copy(g_t, acc.at[iobuf[s]], add=True)

        zero_acc()
        plsc.subcore_barrier()
        pipe(c_issue, c_proc)
        plsc.subcore_barrier()
        flush_acc(mp_hbm, chunk * N)
        plsc.subcore_barrier()


@functools.cache
def _s2_kernel():
    return pl.kernel(
        _s2_body,
        mesh=_sc_mesh(),
        out_type=[jax.ShapeDtypeStruct((NC * N, BAND), jnp.float32),
                  jax.ShapeDtypeStruct((NCHUNK * N, 2 * DC), jnp.float32)],
        scratch_types=[
            pltpu.VMEM((NTH * GR2,), jnp.int32),
            pltpu.VMEM((NTH * GR2,), jnp.int32),
            pltpu.VMEM((GR2,), jnp.int32),
            pltpu.VMEM((GR2,), jnp.int32),
            pltpu.VMEM((GR2,), jnp.int32),
            pltpu.VMEM((GR2,), jnp.int32),
            pltpu.VMEM((GR2, BAND), jnp.float32),
            pltpu.VMEM((GR2, BAND), jnp.float32),
            pltpu.VMEM((GR2, BAND), jnp.float32),
            pltpu.VMEM((GR2, BAND), jnp.float32),
            pltpu.VMEM_SHARED((N, BAND), jnp.float32),
            pltpu.SemaphoreType.DMA,
            pltpu.SemaphoreType.DMA,
            pltpu.SemaphoreType.DMA,
            pltpu.SemaphoreType.DMA,
        ],
    )


def _s2(w, sip, rjp, VY4):
    return _s2_kernel()(w, sip, rjp, VY4)


def kernel(h, p, e, senders, receivers, snorm_n, WA, bA, WB, bB, WC, bC,
           WU, bU, WV, bV, WX, bX, WY, bY, gamma_e, beta_e, gamma_n, beta_n):
    bA2 = bA.reshape(1, D)
    bB2 = bB.reshape(1, D)
    bC2 = bC.reshape(1, D)
    bU2 = bU.reshape(1, D)
    bV2 = bV.reshape(1, D)
    bX2 = bX.reshape(1, D)
    bY2 = bY.reshape(1, D)
    WU1, WU2 = WU[:D], WU[D:]
    WV1, WV2 = WV[:D], WV[D:]

    A, B, U, X, VY4 = _t1a(h, p, WA, bA2, WB, bB2, WU1, WU2, bU2,
                           WV1, WV2, bV2, WX, bX2, WY, bY2)
    C = _t1b(e, WC, bC2)
    eta = _s1(A, B, C, senders, receivers)

    s, q = _t1c(eta)
    mean = s / E
    var = q / E - mean * mean
    rs = lax.rsqrt(var + 1e-5)
    scale = gamma_e.reshape(1, D) * rs
    shift = beta_e.reshape(1, D) - mean * scale

    e_out, w = _t1d(eta, e, scale, shift)

    pad = NGPAD * GR2 - E
    sip = jnp.pad(senders, (0, pad))
    rjp = jnp.pad(receivers, (0, pad))
    wsumB, mp4 = _s2(w, sip, rjp, VY4.reshape(NCHUNK * N, 2 * DC))

    snorm2 = snorm_n.reshape(N, 1)
    npre, p_out, ns, nq = _t2a(U, mp4.reshape(NCHUNK, N, 2 * DC),
                               wsumB.reshape(NC, N, BAND), X, p, snorm2)
    mean2 = ns / N
    var2 = nq / N - mean2 * mean2
    rs2 = lax.rsqrt(var2 + 1e-5)
    scale2 = gamma_n.reshape(1, D) * rs2
    shift2 = beta_n.reshape(1, D) - mean2 * scale2
    h_out = _t2b(h, npre, scale2, shift2)

    return (h_out, p_out, e_out)


# speedup vs baseline: 2.7609x; 1.1535x over previous
import functools

import jax
import jax.numpy as jnp
from jax import lax
from jax.experimental import pallas as pl
from jax.experimental.pallas import tpu as pltpu
from jax.experimental.pallas import tpu_sc as plsc

N = 10000
E = 160000
D = 256
LANES = 16
NC = 2
NS = 16
NW = NC * NS
BAND = D // NC
DC = 64
NCHUNK = D // DC
GROUP = 128
NGROUPS = E // GROUP
RPT = 632
RPT_LAST = N - 15 * RPT
NBLK = 1000
EBLK = 2000


def _t1a_body(h_ref, p_ref, wa, ba, wb, bb, wu1, wu2, bu, wv1, wv2, bv,
              wx, bx, wy, by,
              a_out, b_out, u_out, x_out, vy4_out):
    h = h_ref[...]
    p = p_ref[...]
    f32 = jnp.float32
    a_out[...] = jnp.dot(h, wa[...], preferred_element_type=f32) + ba[...]
    b_out[...] = jnp.dot(h, wb[...], preferred_element_type=f32) + bb[...]
    u_out[...] = (jnp.dot(h, wu1[...], preferred_element_type=f32)
                  + jnp.dot(p, wu2[...], preferred_element_type=f32) + bu[...])
    x_out[...] = jnp.dot(p, wx[...], preferred_element_type=f32) + bx[...]
    v = (jnp.dot(h, wv1[...], preferred_element_type=f32)
         + jnp.dot(p, wv2[...], preferred_element_type=f32) + bv[...])
    y = jnp.dot(p, wy[...], preferred_element_type=f32) + by[...]
    for c in range(NCHUNK):
        vy4_out[c] = jnp.concatenate(
            [v[:, c * DC:(c + 1) * DC], y[:, c * DC:(c + 1) * DC]], axis=1)


def _t1a(h, p, WA, bA, WB, bB, WU1, WU2, bU, WV1, WV2, bV, WX, bX, WY, bY):
    row = pl.BlockSpec((NBLK, D), lambda n: (n, 0))
    wspec = pl.BlockSpec((D, D), lambda n: (0, 0))
    bspec = pl.BlockSpec((1, D), lambda n: (0, 0))
    vyspec = pl.BlockSpec((NCHUNK, NBLK, 2 * DC), lambda n: (0, n, 0))
    f32 = jnp.float32
    return pl.pallas_call(
        _t1a_body,
        grid=(N // NBLK,),
        in_specs=[row, row, wspec, bspec, wspec, bspec, wspec, wspec, bspec,
                  wspec, wspec, bspec, wspec, bspec, wspec, bspec],
        out_specs=[row, row, row, row, vyspec],
        out_shape=[jax.ShapeDtypeStruct((N, D), f32)] * 4
        + [jax.ShapeDtypeStruct((NCHUNK, N, 2 * DC), f32)],
    )(h, p, WA, bA, WB, bB, WU1, WU2, bU, WV1, WV2, bV, WX, bX, WY, bY)


def _t1b_body(e_ref, wc, bc, c_out):
    c_out[...] = (jnp.dot(e_ref[...], wc[...], preferred_element_type=jnp.float32)
                  + bc[...])


def _t1b(e, WC, bC):
    return pl.pallas_call(
        _t1b_body,
        grid=(E // EBLK,),
        in_specs=[pl.BlockSpec((EBLK, D), lambda n: (n, 0)),
                  pl.BlockSpec((D, D), lambda n: (0, 0)),
                  pl.BlockSpec((1, D), lambda n: (0, 0))],
        out_specs=pl.BlockSpec((EBLK, D), lambda n: (n, 0)),
        out_shape=jax.ShapeDtypeStruct((E, D), jnp.float32),
    )(e, WC, bC)


def _t1c_body(eta_ref, s_out, q_out):
    x = eta_ref[...].astype(jnp.float32)
    ps = jnp.sum(x, axis=0, keepdims=True)
    pq = jnp.sum(x * x, axis=0, keepdims=True)

    @pl.when(pl.program_id(0) == 0)
    def _():
        s_out[...] = jnp.zeros_like(s_out)
        q_out[...] = jnp.zeros_like(q_out)

    s_out[...] += ps
    q_out[...] += pq


def _t1c(eta):
    f32 = jnp.float32
    acc = pl.BlockSpec((1, D), lambda n: (0, 0))
    return pl.pallas_call(
        _t1c_body,
        grid=(E // EBLK,),
        in_specs=[pl.BlockSpec((EBLK, D), lambda n: (n, 0))],
        out_specs=[acc, acc],
        out_shape=[jax.ShapeDtypeStruct((1, D), f32)] * 2,
    )(eta)


def _t1d_body(eta_ref, e_ref, sc_ref, sh_ref, eout_out, w_out):
    eta = eta_ref[...].astype(jnp.float32)
    eo = e_ref[...] + jnp.maximum(eta * sc_ref[...] + sh_ref[...], 0.0)
    eout_out[...] = eo
    w_out[...] = 1.0 / (1.0 + jnp.exp(-eo))


def _t1d(eta, e, scale, shift):
    f32 = jnp.float32
    row = pl.BlockSpec((EBLK, D), lambda n: (n, 0))
    vec = pl.BlockSpec((1, D), lambda n: (0, 0))
    return pl.pallas_call(
        _t1d_body,
        grid=(E // EBLK,),
        in_specs=[row, row, vec, vec],
        out_specs=[row, row],
        out_shape=[jax.ShapeDtypeStruct((E, D), f32)] * 2,
    )(eta, e, scale, shift)


def _t2a_body(u_ref, mp_ref, wsum_ref, x_ref, p_ref, sn_ref,
              npre_out, pout_out, s_out, q_out):
    agg = jnp.concatenate([mp_ref[c][:, :DC] for c in range(NCHUNK)], axis=1)
    pagg = jnp.concatenate([mp_ref[c][:, DC:] for c in range(NCHUNK)], axis=1)
    wsum = jnp.concatenate([wsum_ref[b] for b in range(NC)], axis=1)
    inv = 1.0 / (wsum + 1e-6)
    nf = (u_ref[...] + agg * inv) * sn_ref[...]
    npre_out[...] = nf
    pout_out[...] = p_ref[...] + jnp.tanh(x_ref[...] + pagg * inv)

    @pl.when(pl.program_id(0) == 0)
    def _():
        s_out[...] = jnp.zeros_like(s_out)
        q_out[...] = jnp.zeros_like(q_out)

    s_out[...] += jnp.sum(nf, axis=0, keepdims=True)
    q_out[...] += jnp.sum(nf * nf, axis=0, keepdims=True)


def _t2a(U, mp4, wsumB, X, p, snorm2):
    f32 = jnp.float32
    row = pl.BlockSpec((NBLK, D), lambda n: (n, 0))
    mpspec = pl.BlockSpec((NCHUNK, NBLK, 2 * DC), lambda n: (0, n, 0))
    band = pl.BlockSpec((NC, NBLK, BAND), lambda n: (0, n, 0))
    acc = pl.BlockSpec((1, D), lambda n: (0, 0))
    return pl.pallas_call(
        _t2a_body,
        grid=(N // NBLK,),
        in_specs=[row, mpspec, band, row, row,
                  pl.BlockSpec((NBLK, 1), lambda n: (n, 0))],
        out_specs=[row, row, acc, acc],
        out_shape=[jax.ShapeDtypeStruct((N, D), f32)] * 2
        + [jax.ShapeDtypeStruct((1, D), f32)] * 2,
    )(U, mp4, wsumB, X, p, snorm2)


def _t2b_body(h_ref, npre_ref, sc_ref, sh_ref, hout_out):
    nf = jnp.maximum(npre_ref[...] * sc_ref[...] + sh_ref[...], 0.0)
    hout_out[...] = h_ref[...] + nf


def _t2b(h, npre, scale2, shift2):
    row = pl.BlockSpec((NBLK, D), lambda n: (n, 0))
    vec = pl.BlockSpec((1, D), lambda n: (0, 0))
    return pl.pallas_call(
        _t2b_body,
        grid=(N // NBLK,),
        in_specs=[row, row, vec, vec],
        out_specs=row,
        out_shape=jax.ShapeDtypeStruct((N, D), jnp.float32),
    )(h, npre, scale2, shift2)


@functools.cache
def _sc_mesh():
    return plsc.VectorSubcoreMesh(core_axis_name="c", subcore_axis_name="s")


G1 = 64
NG1 = E // G1
NT1 = 80
NTH1 = NT1 // 2


def _s1_body(a_hbm, b_hbm, c_hbm, si_hbm, rj_hbm, eta_hbm,
             i_all, j_all, a0, a1, b0, b1, c0, c1,
             sem_a0, sem_a1, sem_b0, sem_b1, sem_c0, sem_c1):
    cid = lax.axis_index("c")
    sid = lax.axis_index("s")
    wid = sid * NC + cid
    abuf = (a0, a1)
    bbuf = (b0, b1)
    cbuf = (c0, c1)
    sem_as = (sem_a0, sem_a1)
    sem_bs = (sem_b0, sem_b1)
    sem_cs = (sem_c0, sem_c1)

    def _crows(lo_h, t):
        off = pl.multiple_of((lo_h + t) * G1, 8)
        return c_hbm.at[pl.ds(off, G1)]

    def issue(t, lo_h, s):
        isl = i_all.at[pl.ds(t * G1, G1)]
        jsl = j_all.at[pl.ds(t * G1, G1)]
        pltpu.async_copy(a_hbm.at[isl], abuf[s], sem_as[s])
        pltpu.async_copy(b_hbm.at[jsl], bbuf[s], sem_bs[s])
        pltpu.async_copy(_crows(lo_h, t), cbuf[s], sem_cs[s])

    def proc(t, lo_h, s):
        a_t, b_t, c_t = abuf[s], bbuf[s], cbuf[s]
        isl = i_all.at[pl.ds(t * G1, G1)]
        jsl = j_all.at[pl.ds(t * G1, G1)]
        pltpu.make_async_copy(a_hbm.at[isl], a_t, sem_as[s]).wait()
        pltpu.make_async_copy(b_hbm.at[jsl], b_t, sem_bs[s]).wait()
        pltpu.make_async_copy(_crows(lo_h, t), c_t, sem_cs[s]).wait()

        def row_body(r, c2):
            for u in range(D // LANES):
                sl = pl.ds(u * LANES, LANES)
                c_t[r, sl] = a_t[r, sl] + b_t[r, sl] + c_t[r, sl]
            return c2

        lax.fori_loop(0, G1, row_body, 0)
        pltpu.sync_copy(c_t, eta_hbm.at[pl.ds(
            pl.multiple_of((lo_h + t) * G1, 8), G1)])

    for hh in range(2):
        lo_h = wid * NT1 + hh * NTH1
        cnt_h = jnp.minimum(NTH1, NG1 - lo_h)
        ioff = pl.multiple_of(lo_h * G1, 8)
        pltpu.sync_copy(si_hbm.at[pl.ds(ioff, NTH1 * G1)], i_all)
        pltpu.sync_copy(rj_hbm.at[pl.ds(ioff, NTH1 * G1)], j_all)

        @pl.when(0 < cnt_h)
        def _():
            issue(0, lo_h, 0)

        def pair(tp, carry):
            t0 = 2 * tp

            @pl.when(t0 + 1 < cnt_h)
            def _():
                issue(t0 + 1, lo_h, 1)

            @pl.when(t0 < cnt_h)
            def _():
                proc(t0, lo_h, 0)

            @pl.when(t0 + 2 < cnt_h)
            def _():
                issue(t0 + 2, lo_h, 0)

            @pl.when(t0 + 1 < cnt_h)
            def _():
                proc(t0 + 1, lo_h, 1)

            return carry

        lax.fori_loop(0, NTH1 // 2, pair, 0)


@functools.cache
def _s1_kernel():
    f32 = jnp.float32
    return pl.kernel(
        _s1_body,
        mesh=_sc_mesh(),
        out_type=jax.ShapeDtypeStruct((E, D), f32),
        scratch_types=[
            pltpu.VMEM((NTH1 * G1,), jnp.int32),
            pltpu.VMEM((NTH1 * G1,), jnp.int32),
            pltpu.VMEM((G1, D), f32),
            pltpu.VMEM((G1, D), f32),
            pltpu.VMEM((G1, D), f32),
            pltpu.VMEM((G1, D), f32),
            pltpu.VMEM((G1, D), f32),
            pltpu.VMEM((G1, D), f32),
            pltpu.SemaphoreType.DMA,
            pltpu.SemaphoreType.DMA,
            pltpu.SemaphoreType.DMA,
            pltpu.SemaphoreType.DMA,
            pltpu.SemaphoreType.DMA,
            pltpu.SemaphoreType.DMA,
        ],
    )


def _s1(A, B, C, sip, rjp):
    return _s1_kernel()(A, B, C, sip, rjp)


GR2 = 64
NG2 = E // GR2
NT2 = 160
NTH = NT2 // 2
NGPAD = NT2 * NS


def _s2_body(w_hbm, si_hbm, rj_hbm, vy4_hbm,
             wsum_hbm, mp_hbm,
             i_all, j_all, io0, io1, jo0, jo1, wc0, wc1, g0, g1, acc,
             sem_g0, sem_g1, sem_w0, sem_w1):
    cid = lax.axis_index("c")
    sid = lax.axis_index("s")
    col0 = cid * BAND
    zero16 = jnp.zeros((LANES,), jnp.float32)

    def load_idx(lo_h):
        off = pl.multiple_of(lo_h * GR2, 8)
        pltpu.sync_copy(si_hbm.at[pl.ds(off, NTH * GR2)], i_all)
        pltpu.sync_copy(rj_hbm.at[pl.ds(off, NTH * GR2)], j_all)

    def stage_idx(t, io):
        for q in range(GR2 // LANES):
            sl = pl.ds(q * LANES, LANES)
            io[sl] = i_all[pl.ds(t * GR2 + q * LANES, LANES)]

    def zero_acc():
        def zrow(r, carry):
            for u in range(BAND // LANES):
                g0[r, pl.ds(u * LANES, LANES)] = zero16
            return carry

        lax.fori_loop(0, GR2, zrow, 0)

        @pl.when(sid < 15)
        def _():
            base = sid * RPT
            for k in range(9):
                pltpu.sync_copy(g0, acc.at[pl.ds(base + k * GR2, GR2)])
            pltpu.sync_copy(g0.at[pl.ds(0, RPT - 9 * GR2)],
                            acc.at[pl.ds(base + 9 * GR2, RPT - 9 * GR2)])

        @pl.when(sid == 15)
        def _():
            base = 15 * RPT
            for k in range(8):
                pltpu.sync_copy(g0, acc.at[pl.ds(base + k * GR2, GR2)])
            pltpu.sync_copy(
                g0.at[pl.ds(0, RPT_LAST - 8 * GR2)],
                acc.at[pl.ds(base + 8 * GR2, RPT_LAST - 8 * GR2)])

    def flush_acc(dst_hbm, dbase):
        @pl.when(sid < 15)
        def _():
            ro = sid * RPT
            pltpu.sync_copy(acc.at[pl.ds(ro, RPT)],
                            dst_hbm.at[pl.ds(dbase + ro, RPT)])

        @pl.when(sid == 15)
        def _():
            ro = 15 * RPT
            pltpu.sync_copy(acc.at[pl.ds(ro, RPT_LAST)],
                            dst_hbm.at[pl.ds(dbase + ro, RPT_LAST)])

    def _wband(lo_h, t):
        off = pl.multiple_of((lo_h + t) * GR2, 8)
        return w_hbm.at[pl.ds(off, GR2), pl.ds(col0, BAND)]

    def pipe(issue, proc):
        for hh in range(2):
            lo_h = sid * NT2 + hh * NTH
            cnt_h = jnp.minimum(NTH, NG2 - lo_h)
            load_idx(lo_h)

            @pl.when(0 < cnt_h)
            def _():
                issue(0, lo_h, 0)

            def pair(tp, carry):
                t0 = 2 * tp

                @pl.when(t0 + 1 < cnt_h)
                def _():
                    issue(t0 + 1, lo_h, 1)

                @pl.when(t0 < cnt_h)
                def _():
                    proc(t0, lo_h, 0)

                @pl.when(t0 + 2 < cnt_h)
                def _():
                    issue(t0 + 2, lo_h, 0)

                @pl.when(t0 + 1 < cnt_h)
                def _():
                    proc(t0 + 1, lo_h, 1)

                return carry

            lax.fori_loop(0, NTH // 2, pair, 0)

    gbuf = (g0, g1)
    wbuf = (wc0, wc1)
    iobuf = (io0, io1)
    jobuf = (jo0, jo1)
    sem_gs = (sem_g0, sem_g1)
    sem_ws = (sem_w0, sem_w1)

    def w_issue(t, lo_h, s):
        pltpu.async_copy(_wband(lo_h, t), gbuf[s], sem_ws[s])

    def w_proc(t, lo_h, s):
        pltpu.make_async_copy(_wband(lo_h, t), gbuf[s], sem_ws[s]).wait()
        stage_idx(t, iobuf[s])
        pltpu.sync_copy(gbuf[s], acc.at[iobuf[s]], add=True)

    zero_acc()
    plsc.subcore_barrier()
    pipe(w_issue, w_proc)
    plsc.subcore_barrier()
    flush_acc(wsum_hbm, cid * N)
    plsc.subcore_barrier()

    for ch in range(NCHUNK // NC):
        chunk = cid * (NCHUNK // NC) + ch

        def c_issue(t, lo_h, s, _ch=ch):
            jo = jobuf[s]
            for q in range(GR2 // LANES):
                sl = pl.ds(q * LANES, LANES)
                jo[sl] = (j_all[pl.ds(t * GR2 + q * LANES, LANES)]
                          + (cid * (NCHUNK // NC) + _ch) * N)
            pltpu.async_copy(vy4_hbm.at[jo], gbuf[s], sem_gs[s])
            pltpu.async_copy(_wband(lo_h, t), wbuf[s], sem_ws[s])

        def c_proc(t, lo_h, s, _ch=ch):
            g_t = gbuf[s]
            wc = wbuf[s]
            pltpu.make_async_copy(vy4_hbm.at[jobuf[s]], g_t,
                                  sem_gs[s]).wait()
            pltpu.make_async_copy(_wband(lo_h, t), wc, sem_ws[s]).wait()

            def row_body(r, c2):
                for u in range(DC // LANES):
                    sl = pl.ds(u * LANES, LANES)
                    sr = pl.ds(DC + u * LANES, LANES)
                    w = wc[r, pl.ds(_ch * DC + u * LANES, LANES)]
                    g_t[r, sl] = g_t[r, sl] * w
                    g_t[r, sr] = g_t[r, sr] * w
                return c2

            lax.fori_loop(0, GR2, row_body, 0)
            stage_idx(t, iobuf[s])
            pltpu.sync_copy(g_t, acc.at[iobuf[s]], add=True)

        zero_acc()
        plsc.subcore_barrier()
        pipe(c_issue, c_proc)
        plsc.subcore_barrier()
        flush_acc(mp_hbm, chunk * N)
        plsc.subcore_barrier()


@functools.cache
def _s2_kernel():
    return pl.kernel(
        _s2_body,
        mesh=_sc_mesh(),
        out_type=[jax.ShapeDtypeStruct((NC * N, BAND), jnp.float32),
                  jax.ShapeDtypeStruct((NCHUNK * N, 2 * DC), jnp.float32)],
        scratch_types=[
            pltpu.VMEM((NTH * GR2,), jnp.int32),
            pltpu.VMEM((NTH * GR2,), jnp.int32),
            pltpu.VMEM((GR2,), jnp.int32),
            pltpu.VMEM((GR2,), jnp.int32),
            pltpu.VMEM((GR2,), jnp.int32),
            pltpu.VMEM((GR2,), jnp.int32),
            pltpu.VMEM((GR2, BAND), jnp.float32),
            pltpu.VMEM((GR2, BAND), jnp.float32),
            pltpu.VMEM((GR2, BAND), jnp.float32),
            pltpu.VMEM((GR2, BAND), jnp.float32),
            pltpu.VMEM_SHARED((N, BAND), jnp.float32),
            pltpu.SemaphoreType.DMA,
            pltpu.SemaphoreType.DMA,
            pltpu.SemaphoreType.DMA,
            pltpu.SemaphoreType.DMA,
        ],
    )


def _s2(w, sip, rjp, VY4):
    return _s2_kernel()(w, sip, rjp, VY4)


def kernel(h, p, e, senders, receivers, snorm_n, WA, bA, WB, bB, WC, bC,
           WU, bU, WV, bV, WX, bX, WY, bY, gamma_e, beta_e, gamma_n, beta_n):
    bA2 = bA.reshape(1, D)
    bB2 = bB.reshape(1, D)
    bC2 = bC.reshape(1, D)
    bU2 = bU.reshape(1, D)
    bV2 = bV.reshape(1, D)
    bX2 = bX.reshape(1, D)
    bY2 = bY.reshape(1, D)
    WU1, WU2 = WU[:D], WU[D:]
    WV1, WV2 = WV[:D], WV[D:]

    A, B, U, X, VY4 = _t1a(h, p, WA, bA2, WB, bB2, WU1, WU2, bU2,
                           WV1, WV2, bV2, WX, bX2, WY, bY2)
    C = _t1b(e, WC, bC2)
    pad = NGPAD * GR2 - E
    sip = jnp.pad(senders, (0, pad))
    rjp = jnp.pad(receivers, (0, pad))
    eta = _s1(A, B, C, sip, rjp)

    s, q = _t1c(eta)
    mean = s / E
    var = q / E - mean * mean
    rs = lax.rsqrt(var + 1e-5)
    scale = gamma_e.reshape(1, D) * rs
    shift = beta_e.reshape(1, D) - mean * scale

    e_out, w = _t1d(eta, e, scale, shift)

    wsumB, mp4 = _s2(w, sip, rjp, VY4.reshape(NCHUNK * N, 2 * DC))

    snorm2 = snorm_n.reshape(N, 1)
    npre, p_out, ns, nq = _t2a(U, mp4.reshape(NCHUNK, N, 2 * DC),
                               wsumB.reshape(NC, N, BAND), X, p, snorm2)
    mean2 = ns / N
    var2 = nq / N - mean2 * mean2
    rs2 = lax.rsqrt(var2 + 1e-5)
    scale2 = gamma_n.reshape(1, D) * rs2
    shift2 = beta_n.reshape(1, D) - mean2 * scale2
    h_out = _t2b(h, npre, scale2, shift2)

    return (h_out, p_out, e_out)
